# baseline (speedup 1.0000x reference)
import functools

import jax
import jax.numpy as jnp
from jax import lax
from jax.experimental import pallas as pl
from jax.experimental.pallas import tpu as pltpu
from jax.experimental.pallas import tpu_sc as plsc

NC = 2
NS = 16
CW = 128
DW = 16
HD = 64


def _sc_aggregate(fhalves, src_r, dst_r, zeros_acc, zeros_deg, ones_deg,
                  n, n_pad, cpw):
    rows_per_sub = n_pad // NS
    mesh = plsc.VectorSubcoreMesh(core_axis_name="c", subcore_axis_name="s")

    @functools.partial(
        pl.kernel,
        out_type=(
            jax.ShapeDtypeStruct((NC, n_pad, HD), jnp.float32),
            jax.ShapeDtypeStruct((n_pad, DW), jnp.float32),
        ),
        mesh=mesh,
        compiler_params=pltpu.CompilerParams(use_tc_tiling_on_sc=False),
        scratch_types=(
            pltpu.VMEM((cpw + 2, CW), jnp.int32),
            pltpu.VMEM((cpw, CW), jnp.int32),
            pltpu.VMEM((CW, HD), jnp.float32),
            pltpu.VMEM((CW, HD), jnp.float32),
            pltpu.VMEM((CW, DW), jnp.float32),
            pltpu.VMEM_SHARED((n_pad, HD), jnp.float32),
            pltpu.VMEM_SHARED((n_pad, DW), jnp.float32),
            pltpu.SemaphoreType.DMA,
            pltpu.SemaphoreType.DMA,
        ),
    )
    def k(feat_hbm, src_hbm, dst_hbm, zacc_hbm, zdeg_hbm, ones_hbm,
          pagg_hbm, deg_hbm, src_v, dst_v, rows0, rows1, ones_v, acc, dacc,
          sem0, sem1):
        c = lax.axis_index("c")
        s = lax.axis_index("s")
        r0 = pl.multiple_of(s * rows_per_sub, 8)
        pltpu.sync_copy(zacc_hbm.at[pl.ds(r0, rows_per_sub)],
                        acc.at[pl.ds(r0, rows_per_sub)])
        pltpu.sync_copy(zdeg_hbm.at[pl.ds(r0, rows_per_sub)],
                        dacc.at[pl.ds(r0, rows_per_sub)])
        pltpu.sync_copy(ones_hbm, ones_v)
        pltpu.sync_copy(src_hbm.at[c, s], src_v.at[pl.ds(0, cpw)])
        pltpu.sync_copy(dst_hbm.at[c, s], dst_v)
        zero16 = jnp.zeros((16,), jnp.int32)
        for extra in (cpw, cpw + 1):
            for q in range(CW // 16):
                src_v[extra, pl.ds(q * 16, 16)] = zero16
        plsc.subcore_barrier()

        count_deg = c == 0

        def gather(j, buf, sem):
            pltpu.async_copy(feat_hbm.at[src_v.at[j]], buf, sem)

        def wait(buf, sem):
            pltpu.make_async_copy(feat_hbm.at[src_v.at[0]], buf, sem).wait()

        def scatter(j, buf):
            @pl.when(count_deg)
            def _():
                pltpu.sync_copy(ones_v, dacc.at[dst_v.at[j]], add=True)

            pltpu.sync_copy(buf, acc.at[dst_v.at[j]], add=True)

        gather(0, rows0, sem0)

        def body(p, carry):
            j = 2 * p
            gather(j + 1, rows1, sem1)
            wait(rows0, sem0)
            scatter(j, rows0)
            gather(j + 2, rows0, sem0)
            wait(rows1, sem1)
            scatter(j + 1, rows1)
            return carry

        lax.fori_loop(0, cpw // 2, body, 0)
        wait(rows0, sem0)
        plsc.subcore_barrier()
        pltpu.sync_copy(acc.at[pl.ds(r0, rows_per_sub)],
                        pagg_hbm.at[c, pl.ds(r0, rows_per_sub)])

        @pl.when(count_deg)
        def _():
            pltpu.sync_copy(dacc.at[pl.ds(r0, rows_per_sub)],
                            deg_hbm.at[pl.ds(r0, rows_per_sub)])

    return k(fhalves, src_r, dst_r, zeros_acc, zeros_deg, ones_deg)


def _tc_finish(pagg, deg, W, n, blk):
    def body(pagg_ref, deg_ref, w_ref, out_ref):
        d = deg_ref[...]
        rowsum = jnp.sum(d, axis=1, keepdims=True)
        scale = float(DW) / jnp.maximum(rowsum, float(DW))
        h0 = pagg_ref[0] * scale
        h1 = pagg_ref[1] * scale
        out_ref[...] = (
            lax.dot_general(h0, w_ref[:, :HD], (((1,), (1,)), ((), ())),
                            preferred_element_type=jnp.float32)
            + lax.dot_general(h1, w_ref[:, HD:], (((1,), (1,)), ((), ())),
                              preferred_element_type=jnp.float32))

    return pl.pallas_call(
        body,
        grid=(n // blk,),
        in_specs=[
            pl.BlockSpec((NC, blk, HD), lambda i: (0, i, 0)),
            pl.BlockSpec((blk, DW), lambda i: (i, 0)),
            pl.BlockSpec((128, 128), lambda i: (0, 0)),
        ],
        out_specs=pl.BlockSpec((blk, 128), lambda i: (i, 0)),
        out_shape=jax.ShapeDtypeStruct((n, 128), jnp.float32),
    )(pagg, deg, W)


def kernel(feature, edge_index, W):
    n, _ = feature.shape
    e = edge_index.shape[1]
    src = edge_index[0].astype(jnp.int32)
    dst = edge_index[1].astype(jnp.int32)

    cpw = -(-e // (NS * CW))
    cpw += cpw % 2
    e_cap = NS * cpw * CW
    n_pad = -(-(n + 1) // (NS * 8)) * (NS * 8)

    fhalves = jnp.concatenate([feature[:, :HD], feature[:, HD:]], axis=0)

    pad = e_cap - e
    src_pad = jnp.concatenate([src, jnp.zeros((pad,), jnp.int32)])
    dst_pad = jnp.concatenate([dst, jnp.full((pad,), n, jnp.int32)])
    src_r = jnp.stack([src_pad, src_pad + n]).reshape(NC, NS, cpw, CW)
    dst_r = jnp.broadcast_to(dst_pad, (NC, e_cap)).reshape(NC, NS, cpw, CW)

    zeros_acc = jnp.zeros((n_pad, HD), jnp.float32)
    zeros_deg = jnp.zeros((n_pad, DW), jnp.float32)
    ones_deg = jnp.ones((CW, DW), jnp.float32)

    pagg, deg = _sc_aggregate(fhalves, src_r, dst_r, zeros_acc, zeros_deg,
                              ones_deg, n, n_pad, cpw)
    return _tc_finish(pagg, deg, W, n, 1000)


# speedup vs baseline: 1.2020x; 1.2020x over previous
import functools

import jax
import jax.numpy as jnp
from jax import lax
from jax.experimental import pallas as pl
from jax.experimental.pallas import tpu as pltpu
from jax.experimental.pallas import tpu_sc as plsc

NC = 2
NS = 16
CW = 128
DW = 16


def _sc_aggregate(fb, src_r, dst_r, zeros_acc, zeros_deg, ones_deg,
                  n, n_pad, cpw):
    rows_per_sub = n_pad // NS
    mesh = plsc.VectorSubcoreMesh(core_axis_name="c", subcore_axis_name="s")

    @functools.partial(
        pl.kernel,
        out_type=(
            jax.ShapeDtypeStruct((NC, n_pad, 128), jnp.bfloat16),
            jax.ShapeDtypeStruct((NC, n_pad, DW), jnp.float32),
        ),
        mesh=mesh,
        compiler_params=pltpu.CompilerParams(use_tc_tiling_on_sc=False),
        scratch_types=(
            pltpu.VMEM((cpw, CW), jnp.int32),
            pltpu.VMEM((cpw, CW), jnp.int32),
            pltpu.VMEM((CW, 128), jnp.bfloat16),
            pltpu.VMEM((CW, DW), jnp.float32),
            pltpu.VMEM_SHARED((n_pad, 128), jnp.bfloat16),
            pltpu.VMEM_SHARED((n_pad, DW), jnp.float32),
            pltpu.SemaphoreType.DMA,
        ),
    )
    def k(feat_hbm, src_hbm, dst_hbm, zacc_hbm, zdeg_hbm, ones_hbm,
          pagg_hbm, deg_hbm, src_v, dst_v, rows_v, ones_v, acc, dacc, sem):
        c = lax.axis_index("c")
        s = lax.axis_index("s")
        r0 = pl.multiple_of(s * rows_per_sub, 8)
        pltpu.sync_copy(zacc_hbm.at[pl.ds(r0, rows_per_sub)],
                        acc.at[pl.ds(r0, rows_per_sub)])
        pltpu.sync_copy(zdeg_hbm.at[pl.ds(r0, rows_per_sub)],
                        dacc.at[pl.ds(r0, rows_per_sub)])
        pltpu.sync_copy(ones_hbm, ones_v)
        pltpu.sync_copy(src_hbm.at[c, s], src_v)
        pltpu.sync_copy(dst_hbm.at[c, s], dst_v)
        plsc.subcore_barrier()

        def body(j, carry):
            cp = pltpu.async_copy(feat_hbm.at[src_v.at[j]], rows_v, sem)

            pltpu.sync_copy(ones_v, dacc.at[dst_v.at[j]], add=True)

            cp.wait()
            pltpu.sync_copy(rows_v, acc.at[dst_v.at[j]], add=True)
            return carry

        lax.fori_loop(0, cpw, body, 0)
        plsc.subcore_barrier()
        pltpu.sync_copy(acc.at[pl.ds(r0, rows_per_sub)],
                        pagg_hbm.at[c, pl.ds(r0, rows_per_sub)])
        pltpu.sync_copy(dacc.at[pl.ds(r0, rows_per_sub)],
                        deg_hbm.at[c, pl.ds(r0, rows_per_sub)])

    return k(fb, src_r, dst_r, zeros_acc, zeros_deg, ones_deg)


def _tc_finish(pagg, deg, W, n, blk):
    def body(pagg_ref, deg_ref, w_ref, out_ref):
        a = (pagg_ref[0].astype(jnp.float32)
             + pagg_ref[1].astype(jnp.float32))
        d = deg_ref[0] + deg_ref[1]
        rowsum = jnp.sum(d, axis=1, keepdims=True)
        scale = float(DW) / jnp.maximum(rowsum, float(DW))
        h = a * scale
        out_ref[...] = lax.dot_general(
            h, w_ref[...], (((1,), (1,)), ((), ())),
            preferred_element_type=jnp.float32)

    return pl.pallas_call(
        body,
        grid=(n // blk,),
        in_specs=[
            pl.BlockSpec((NC, blk, 128), lambda i: (0, i, 0)),
            pl.BlockSpec((NC, blk, DW), lambda i: (0, i, 0)),
            pl.BlockSpec((128, 128), lambda i: (0, 0)),
        ],
        out_specs=pl.BlockSpec((blk, 128), lambda i: (i, 0)),
        out_shape=jax.ShapeDtypeStruct((n, 128), jnp.float32),
    )(pagg, deg, W)


def kernel(feature, edge_index, W):
    n, _ = feature.shape
    e = edge_index.shape[1]
    src = edge_index[0].astype(jnp.int32)
    dst = edge_index[1].astype(jnp.int32)

    cpw = -(-e // (NC * NS * CW))
    e_cap = NC * NS * cpw * CW
    n_pad = -(-(n + 1) // (NS * 8)) * (NS * 8)

    fb = feature.astype(jnp.bfloat16)

    pad = e_cap - e
    src_r = jnp.concatenate([src, jnp.zeros((pad,), jnp.int32)]
                            ).reshape(NC, NS, cpw, CW)
    dst_r = jnp.concatenate([dst, jnp.full((pad,), n, jnp.int32)]
                            ).reshape(NC, NS, cpw, CW)

    zeros_acc = jnp.zeros((n_pad, 128), jnp.bfloat16)
    zeros_deg = jnp.zeros((n_pad, DW), jnp.float32)
    ones_deg = jnp.ones((CW, DW), jnp.float32)

    pagg, deg = _sc_aggregate(fb, src_r, dst_r, zeros_acc, zeros_deg,
                              ones_deg, n, n_pad, cpw)
    return _tc_finish(pagg, deg, W, n, 1000)
